# baseline (speedup 1.0000x reference)
import functools

import jax
import jax.numpy as jnp
from jax import lax
from jax.experimental import pallas as pl
from jax.experimental.pallas import tpu as pltpu
from jax.experimental.pallas import tpu_sc as plsc

N_ROWS = 100000
NSEG = 512
D = 128
C = 64
NFULL = N_ROWS // C
TAIL = N_ROWS - NFULL * C
TAIL_W = NFULL % 32
NSUB = 16
NW = 32
PAIRS = (NFULL + 2 * NW - 1) // (2 * NW)
SEG_PER_SUB = NSEG // NSUB
SEG_PER_W = NSEG // NW
RV = 16

_mesh = plsc.VectorSubcoreMesh(core_axis_name="c", subcore_axis_name="s")

_ACC_SCRATCH = (
    [pltpu.VMEM((C, D), jnp.float32) for _ in range(8)]
    + [pltpu.VMEM((C,), jnp.int32) for _ in range(2)]
    + [
        pltpu.VMEM((C, D), jnp.float32),
        pltpu.VMEM((TAIL, D), jnp.float32),
        pltpu.VMEM((TAIL, D), jnp.float32),
        pltpu.VMEM((TAIL,), jnp.int32),
        pltpu.VMEM((SEG_PER_SUB, D), jnp.float32),
        pltpu.VMEM((RV, D), jnp.float32),
    ]
    + [pltpu.VMEM((RV, D), jnp.float32) for _ in range(8)]
    + [pltpu.VMEM((RV,), jnp.int32) for _ in range(2)]
    + [pltpu.VMEM_SHARED((NSEG, D), jnp.float32) for _ in range(5)]
    + [pltpu.SemaphoreType.DMA for _ in range(14)]
)


@functools.partial(
    pl.kernel,
    mesh=_mesh,
    out_type=[jax.ShapeDtypeStruct((2, NSEG, D), jnp.float32)] * 5,
    scratch_types=_ACC_SCRATCH,
)
def _sc_accumulate(x0, x1, x2, x3, idx, p0, p1, p2, p3, pc, *scr):
    bufs = [scr[0:2], scr[2:4], scr[4:6], scr[6:8]]
    idx_buf = scr[8:10]
    ones_v, ones_t, x_t, idx_t, z2, c64 = scr[10:16]
    red = [scr[16:18], scr[18:20], scr[20:22], scr[22:24]]
    idx1 = scr[24:26]
    accs = scr[26:30]
    cnt = scr[30]
    sem_l = [scr[31:33], scr[33:35], scr[35:37], scr[37:39]]
    sem_i = scr[39:41]
    sem_s = scr[41:45]

    xs = (x0, x1, x2, x3)
    cid = lax.axis_index("c")
    sid = lax.axis_index("s")
    wid = cid * NSUB + sid
    seg0 = sid * SEG_PER_SUB

    ones16 = jnp.ones((16,), jnp.float32)
    zeros16 = jnp.zeros((16,), jnp.float32)
    c16 = jnp.full((16,), float(C), jnp.float32)

    def fill_ones(i, carry):
        for k in range(D // 16):
            ones_v[i, pl.ds(k * 16, 16)] = ones16
        return carry
    lax.fori_loop(0, C, fill_ones, 0)

    def fill_ones_t(i, carry):
        for k in range(D // 16):
            ones_t[i, pl.ds(k * 16, 16)] = ones16
        return carry
    lax.fori_loop(0, TAIL, fill_ones_t, 0)

    def fill_z(i, carry):
        for k in range(D // 16):
            z2[i, pl.ds(k * 16, 16)] = zeros16
        return carry
    lax.fori_loop(0, SEG_PER_SUB, fill_z, 0)

    def fill_red(i, carry):
        for k in range(D // 16):
            ks = pl.ds(k * 16, 16)
            c64[i, ks] = zeros16
            for a in range(4):
                for q in range(2):
                    red[a][q][i, ks] = zeros16
        return carry
    lax.fori_loop(0, RV, fill_red, 0)
    for k in range(D // 16):
        c64[0, pl.ds(k * 16, 16)] = c16

    for acc in accs:
        pltpu.sync_copy(z2, acc.at[pl.ds(seg0, SEG_PER_SUB)])
    pltpu.sync_copy(z2, cnt.at[pl.ds(seg0, SEG_PER_SUB)])
    plsc.subcore_barrier()

    def start_loads(r, q):
        ch = r * NW + wid
        base = pl.multiple_of(ch * C, C)
        pltpu.async_copy(idx.at[pl.ds(base, C)], idx_buf[q], sem_i[q])
        for a in range(4):
            pltpu.async_copy(xs[a].at[pl.ds(base, C)], bufs[a][q],
                             sem_l[a][q])

    def reduce_rows(buf, out_ref):
        zeros8 = tuple(jnp.zeros((16,), jnp.float32) for _ in range(D // 16))

        def body(i, carry):
            new = []
            for k in range(D // 16):
                ks = pl.ds(k * 16, 16)
                s = carry[k]
                for j in range(4):
                    s = s + buf[4 * i + j, ks]
                new.append(s)
            return tuple(new)

        acc = lax.fori_loop(0, C // 4, body, zeros8)
        for k in range(D // 16):
            out_ref[0, pl.ds(k * 16, 16)] = acc[k]

    def half_round(r, q):
        ch = r * NW + wid
        base = pl.multiple_of(ch * C, C)
        qn = 1 - q

        @pl.when(ch < NFULL)
        def _():
            @pl.when(ch + NW < NFULL)
            def _():
                start_loads(r + 1, qn)

            pltpu.make_async_copy(
                idx.at[pl.ds(base, C)], idx_buf[q], sem_i[q]).wait()
            for a in range(4):
                pltpu.make_async_copy(
                    xs[a].at[pl.ds(base, C)], bufs[a][q], sem_l[a][q]).wait()

            iv = [idx_buf[q][pl.ds(k * 16, 16)] for k in range(C // 16)]
            fast = iv[0][0] == iv[C // 16 - 1][15]

            @pl.when(fast)
            def _():
                idx1[q][...] = iv[0]
                h_s = []
                for a in range(4):
                    reduce_rows(bufs[a][q], red[a][q])
                    h_s.append(
                        pltpu.async_copy(red[a][q], accs[a].at[idx1[q]],
                                         sem_s[a], add=True))
                pltpu.sync_copy(c64, cnt.at[idx1[q]], add=True)
                for h in h_s:
                    h.wait()

            @pl.when(jnp.logical_not(fast))
            def _():
                h_s = []
                for a in range(4):
                    h_s.append(
                        pltpu.async_copy(bufs[a][q], accs[a].at[idx_buf[q]],
                                         sem_s[a], add=True))
                pltpu.sync_copy(ones_v, cnt.at[idx_buf[q]], add=True)
                for h in h_s:
                    h.wait()

    def pair_body(p, carry):
        half_round(2 * p, 0)
        half_round(2 * p + 1, 1)
        return carry

    start_loads(0, 0)
    lax.fori_loop(0, PAIRS, pair_body, 0)

    @pl.when(wid == TAIL_W)
    def _():
        tbase = NFULL * C
        pltpu.sync_copy(idx.at[pl.ds(tbase, TAIL)], idx_t)
        for a in range(4):
            pltpu.sync_copy(xs[a].at[pl.ds(tbase, TAIL)], x_t)
            pltpu.sync_copy(x_t, accs[a].at[idx_t], add=True)
        pltpu.sync_copy(ones_t, cnt.at[idx_t], add=True)

    plsc.subcore_barrier()

    sl = pl.ds(seg0, SEG_PER_SUB)
    for acc, p_out in zip(accs, (p0, p1, p2, p3)):
        pltpu.sync_copy(acc.at[sl], p_out.at[cid, sl])
    pltpu.sync_copy(cnt.at[sl], pc.at[cid, sl])


@functools.partial(
    pl.kernel,
    mesh=_mesh,
    out_type=jax.ShapeDtypeStruct((NSEG, NSEG), jnp.float32),
    scratch_types=[
        pltpu.VMEM((SEG_PER_W, D), jnp.float32),
        pltpu.VMEM((SEG_PER_W, D), jnp.float32),
        pltpu.VMEM((SEG_PER_W, D), jnp.float32),
        pltpu.VMEM((SEG_PER_W, D), jnp.float32),
        pltpu.VMEM((SEG_PER_W, D), jnp.float32),
        pltpu.VMEM((SEG_PER_W, 16), jnp.float32),
    ],
)
def _sc_combine(p0, p1, p2, p3, pc, out,
                a_buf, b_buf, o_buf, ca_buf, cb_buf, rec_buf):
    cid = lax.axis_index("c")
    sid = lax.axis_index("s")
    wid = cid * NSUB + sid
    seg0 = wid * SEG_PER_W
    sl = pl.ds(seg0, SEG_PER_W)

    pltpu.sync_copy(pc.at[0, sl], ca_buf)
    pltpu.sync_copy(pc.at[1, sl], cb_buf)

    def rec_body(i, carry):
        cv = ca_buf[i, pl.ds(0, 16)] + cb_buf[i, pl.ds(0, 16)]
        rec_buf[i, :] = 1.0 / jnp.maximum(cv, 1.0)
        return carry
    lax.fori_loop(0, SEG_PER_W, rec_body, 0)

    def div_body(i, carry):
        recip = rec_buf[i, :]
        for k in range(D // 16):
            ks = pl.ds(k * 16, 16)
            o_buf[i, ks] = (a_buf[i, ks] + b_buf[i, ks]) * recip
        return carry

    for j, p in enumerate((p0, p1, p2, p3)):
        pltpu.sync_copy(p.at[0, sl], a_buf)
        pltpu.sync_copy(p.at[1, sl], b_buf)
        lax.fori_loop(0, SEG_PER_W, div_body, 0)
        pltpu.sync_copy(o_buf, out.at[sl, pl.ds(j * D, D)])


def kernel(x0, x1, x2, x3, index):
    p0, p1, p2, p3, pc = _sc_accumulate(x0, x1, x2, x3, index)
    return _sc_combine(p0, p1, p2, p3, pc)


# speedup vs baseline: 1.0076x; 1.0076x over previous
import functools

import jax
import jax.numpy as jnp
from jax import lax
from jax.experimental import pallas as pl
from jax.experimental.pallas import tpu as pltpu
from jax.experimental.pallas import tpu_sc as plsc

N_ROWS = 100000
NSEG = 512
D = 128
C = 64
NFULL = N_ROWS // C
TAIL = N_ROWS - NFULL * C
TAIL_W = NFULL % 32
NSUB = 16
NW = 32
PAIRS = (NFULL + 2 * NW - 1) // (2 * NW)
SEG_PER_SUB = NSEG // NSUB
SEG_PER_W = NSEG // NW
RV = 16

_mesh = plsc.VectorSubcoreMesh(core_axis_name="c", subcore_axis_name="s")

_ACC_SCRATCH = (
    [pltpu.VMEM((C, D), jnp.float32) for _ in range(8)]
    + [pltpu.VMEM((C,), jnp.int32) for _ in range(2)]
    + [
        pltpu.VMEM((C, D), jnp.float32),
        pltpu.VMEM((TAIL, D), jnp.float32),
        pltpu.VMEM((TAIL, D), jnp.float32),
        pltpu.VMEM((TAIL,), jnp.int32),
        pltpu.VMEM((SEG_PER_SUB, D), jnp.float32),
        pltpu.VMEM((RV, D), jnp.float32),
    ]
    + [pltpu.VMEM((RV, D), jnp.float32) for _ in range(8)]
    + [pltpu.VMEM((RV,), jnp.int32) for _ in range(2)]
    + [pltpu.VMEM_SHARED((NSEG, D), jnp.float32) for _ in range(5)]
    + [pltpu.SemaphoreType.DMA for _ in range(14)]
)


@functools.partial(
    pl.kernel,
    mesh=_mesh,
    out_type=[jax.ShapeDtypeStruct((2, NSEG, D), jnp.float32)] * 5,
    scratch_types=_ACC_SCRATCH,
)
def _sc_accumulate(x0, x1, x2, x3, idx, p0, p1, p2, p3, pc, *scr):
    bufs = [scr[0:2], scr[2:4], scr[4:6], scr[6:8]]
    idx_buf = scr[8:10]
    ones_v, ones_t, x_t, idx_t, z2, c64 = scr[10:16]
    red = [scr[16:18], scr[18:20], scr[20:22], scr[22:24]]
    idx1 = scr[24:26]
    accs = scr[26:30]
    cnt = scr[30]
    sem_l = [scr[31:33], scr[33:35], scr[35:37], scr[37:39]]
    sem_i = scr[39:41]
    sem_s = scr[41:45]

    xs = (x0, x1, x2, x3)
    cid = lax.axis_index("c")
    sid = lax.axis_index("s")
    wid = cid * NSUB + sid
    seg0 = sid * SEG_PER_SUB

    ones16 = jnp.ones((16,), jnp.float32)
    zeros16 = jnp.zeros((16,), jnp.float32)
    c16 = jnp.full((16,), float(C), jnp.float32)

    def fill_ones(i, carry):
        for k in range(D // 16):
            ones_v[i, pl.ds(k * 16, 16)] = ones16
        return carry
    lax.fori_loop(0, C, fill_ones, 0)

    def fill_ones_t(i, carry):
        for k in range(D // 16):
            ones_t[i, pl.ds(k * 16, 16)] = ones16
        return carry
    lax.fori_loop(0, TAIL, fill_ones_t, 0)

    def fill_z(i, carry):
        for k in range(D // 16):
            z2[i, pl.ds(k * 16, 16)] = zeros16
        return carry
    lax.fori_loop(0, SEG_PER_SUB, fill_z, 0)

    def fill_red(i, carry):
        for k in range(D // 16):
            ks = pl.ds(k * 16, 16)
            c64[i, ks] = zeros16
            for a in range(4):
                for q in range(2):
                    red[a][q][i, ks] = zeros16
        return carry
    lax.fori_loop(0, RV, fill_red, 0)
    for k in range(D // 16):
        c64[0, pl.ds(k * 16, 16)] = c16

    for acc in accs:
        pltpu.sync_copy(z2, acc.at[pl.ds(seg0, SEG_PER_SUB)])
    pltpu.sync_copy(z2, cnt.at[pl.ds(seg0, SEG_PER_SUB)])
    plsc.subcore_barrier()

    def start_loads(r, q):
        ch = r * NW + wid
        base = pl.multiple_of(ch * C, C)
        pltpu.async_copy(idx.at[pl.ds(base, C)], idx_buf[q], sem_i[q])
        for a in range(4):
            pltpu.async_copy(xs[a].at[pl.ds(base, C)], bufs[a][q],
                             sem_l[a][q])

    def reduce_rows(buf, out_ref):
        zeros8 = tuple(jnp.zeros((16,), jnp.float32) for _ in range(D // 16))

        def body(i, carry):
            new = []
            for k in range(D // 16):
                ks = pl.ds(k * 16, 16)
                new.append(carry[k] + buf[2 * i, ks] + buf[2 * i + 1, ks])
            return tuple(new)

        acc = lax.fori_loop(0, C // 2, body, zeros8)
        for k in range(D // 16):
            out_ref[0, pl.ds(k * 16, 16)] = acc[k]

    def half_round(r, q):
        ch = r * NW + wid
        base = pl.multiple_of(ch * C, C)
        qn = 1 - q

        @pl.when(ch < NFULL)
        def _():
            @pl.when(ch + NW < NFULL)
            def _():
                start_loads(r + 1, qn)

            pltpu.make_async_copy(
                idx.at[pl.ds(base, C)], idx_buf[q], sem_i[q]).wait()
            for a in range(4):
                pltpu.make_async_copy(
                    xs[a].at[pl.ds(base, C)], bufs[a][q], sem_l[a][q]).wait()

            iv = [idx_buf[q][pl.ds(k * 16, 16)] for k in range(C // 16)]
            fast = iv[0][0] == iv[C // 16 - 1][15]

            @pl.when(fast)
            def _():
                idx1[q][...] = iv[0]
                h_s = []
                for a in range(4):
                    reduce_rows(bufs[a][q], red[a][q])
                    h_s.append(
                        pltpu.async_copy(red[a][q], accs[a].at[idx1[q]],
                                         sem_s[a], add=True))
                pltpu.sync_copy(c64, cnt.at[idx1[q]], add=True)
                for h in h_s:
                    h.wait()

            @pl.when(jnp.logical_not(fast))
            def _():
                h_s = []
                for a in range(4):
                    h_s.append(
                        pltpu.async_copy(bufs[a][q], accs[a].at[idx_buf[q]],
                                         sem_s[a], add=True))
                pltpu.sync_copy(ones_v, cnt.at[idx_buf[q]], add=True)
                for h in h_s:
                    h.wait()

    def pair_body(p, carry):
        half_round(2 * p, 0)
        half_round(2 * p + 1, 1)
        return carry

    start_loads(0, 0)
    lax.fori_loop(0, PAIRS, pair_body, 0)

    @pl.when(wid == TAIL_W)
    def _():
        tbase = NFULL * C
        pltpu.sync_copy(idx.at[pl.ds(tbase, TAIL)], idx_t)
        for a in range(4):
            pltpu.sync_copy(xs[a].at[pl.ds(tbase, TAIL)], x_t)
            pltpu.sync_copy(x_t, accs[a].at[idx_t], add=True)
        pltpu.sync_copy(ones_t, cnt.at[idx_t], add=True)

    plsc.subcore_barrier()

    sl = pl.ds(seg0, SEG_PER_SUB)
    for acc, p_out in zip(accs, (p0, p1, p2, p3)):
        pltpu.sync_copy(acc.at[sl], p_out.at[cid, sl])
    pltpu.sync_copy(cnt.at[sl], pc.at[cid, sl])


@functools.partial(
    pl.kernel,
    mesh=_mesh,
    out_type=jax.ShapeDtypeStruct((NSEG, NSEG), jnp.float32),
    scratch_types=[
        pltpu.VMEM((SEG_PER_W, D), jnp.float32),
        pltpu.VMEM((SEG_PER_W, D), jnp.float32),
        pltpu.VMEM((SEG_PER_W, D), jnp.float32),
        pltpu.VMEM((SEG_PER_W, D), jnp.float32),
        pltpu.VMEM((SEG_PER_W, D), jnp.float32),
        pltpu.VMEM((SEG_PER_W, 16), jnp.float32),
    ],
)
def _sc_combine(p0, p1, p2, p3, pc, out,
                a_buf, b_buf, o_buf, ca_buf, cb_buf, rec_buf):
    cid = lax.axis_index("c")
    sid = lax.axis_index("s")
    wid = cid * NSUB + sid
    seg0 = wid * SEG_PER_W
    sl = pl.ds(seg0, SEG_PER_W)

    pltpu.sync_copy(pc.at[0, sl], ca_buf)
    pltpu.sync_copy(pc.at[1, sl], cb_buf)

    def rec_body(i, carry):
        cv = ca_buf[i, pl.ds(0, 16)] + cb_buf[i, pl.ds(0, 16)]
        rec_buf[i, :] = 1.0 / jnp.maximum(cv, 1.0)
        return carry
    lax.fori_loop(0, SEG_PER_W, rec_body, 0)

    def div_body(i, carry):
        recip = rec_buf[i, :]
        for k in range(D // 16):
            ks = pl.ds(k * 16, 16)
            o_buf[i, ks] = (a_buf[i, ks] + b_buf[i, ks]) * recip
        return carry

    for j, p in enumerate((p0, p1, p2, p3)):
        pltpu.sync_copy(p.at[0, sl], a_buf)
        pltpu.sync_copy(p.at[1, sl], b_buf)
        lax.fori_loop(0, SEG_PER_W, div_body, 0)
        pltpu.sync_copy(o_buf, out.at[sl, pl.ds(j * D, D)])


def kernel(x0, x1, x2, x3, index):
    p0, p1, p2, p3, pc = _sc_accumulate(x0, x1, x2, x3, index)
    return _sc_combine(p0, p1, p2, p3, pc)
